# baseline (speedup 1.0000x reference)
import jax
import jax.numpy as jnp
from jax import lax
from jax.experimental import pallas as pl

K = 5
CLASSES = 1000
BATCH = 16384

_BB = 1024
_NBLK = BATCH // _BB


def _fixed_ind():
    try:
        cpu = jax.local_devices(backend="cpu")[0]
        with jax.default_device(cpu):
            perm = jax.random.permutation(jax.random.key(42), CLASSES - 1)
            return tuple(int(v) for v in perm[:K])
    except Exception:
        return (955, 914, 121, 753, 617)


_IND = _fixed_ind()


def _loss_body(x_ref, y_ref, out_ref):
    x = x_ref[...]
    yb = y_ref[0]

    ci = lax.broadcasted_iota(jnp.int32, (CLASSES, _BB), 0)
    tv = jnp.sum(jnp.where(ci == yb, x, 0.0), axis=0, keepdims=True)

    vals = [tv]
    for j in range(K):
        lo = x[_IND[j]:_IND[j] + 1, :]
        hi = x[_IND[j] + 1:_IND[j] + 2, :]
        vals.append(jnp.where(yb <= _IND[j], hi, lo))
    v = jnp.concatenate(vals, axis=0)

    m = jnp.max(v, axis=0, keepdims=True)
    e = jnp.exp(v - m)
    s = jnp.sum(e, axis=0, keepdims=True)
    p = e / s
    term = jnp.log(p[0:1, :] + 1e-10) + 0.1 * jnp.sum(
        jnp.log((1.0 - p[1:, :]) + 1e-10), axis=0, keepdims=True
    )
    part = -jnp.sum(term) / jnp.float32(BATCH)

    @pl.when(pl.program_id(0) == 0)
    def _init():
        out_ref[...] = jnp.zeros((1, 1), jnp.float32)

    out_ref[...] += jnp.reshape(part, (1, 1))


def kernel(yHat, y):
    xT = yHat.T
    y3 = y.reshape(_NBLK, 1, _BB)
    loss = pl.pallas_call(
        _loss_body,
        grid=(_NBLK,),
        in_specs=[
            pl.BlockSpec((CLASSES, _BB), lambda b: (0, b)),
            pl.BlockSpec((1, 1, _BB), lambda b: (b, 0, 0)),
        ],
        out_specs=pl.BlockSpec((1, 1), lambda b: (0, 0)),
        out_shape=jax.ShapeDtypeStruct((1, 1), jnp.float32),
    )(xT, y3)
    return loss[0, 0]


# speedup vs baseline: 1.1539x; 1.1539x over previous
import jax
import jax.numpy as jnp
from jax import lax
from jax.experimental import pallas as pl

K = 5
CLASSES = 1000
BATCH = 16384

_BB = 2048
_NBLK = BATCH // _BB


def _fixed_ind():
    try:
        cpu = jax.local_devices(backend="cpu")[0]
        with jax.default_device(cpu):
            perm = jax.random.permutation(jax.random.key(42), CLASSES - 1)
            return tuple(int(v) for v in perm[:K])
    except Exception:
        return (955, 914, 121, 753, 617)


_IND = _fixed_ind()


def _loss_body(x_ref, y_ref, out_ref):
    x = x_ref[...]
    yb = y_ref[0]

    ci = lax.broadcasted_iota(jnp.int32, (CLASSES, _BB), 0)
    tv = jnp.sum(jnp.where(ci == yb, x, 0.0), axis=0, keepdims=True)

    vals = [tv]
    for j in range(K):
        lo = x[_IND[j]:_IND[j] + 1, :]
        hi = x[_IND[j] + 1:_IND[j] + 2, :]
        vals.append(jnp.where(yb <= _IND[j], hi, lo))
    v = jnp.concatenate(vals, axis=0)

    m = jnp.max(v, axis=0, keepdims=True)
    e = jnp.exp(v - m)
    s = jnp.sum(e, axis=0, keepdims=True)
    p = e / s
    term = jnp.log(p[0:1, :] + 1e-10) + 0.1 * jnp.sum(
        jnp.log((1.0 - p[1:, :]) + 1e-10), axis=0, keepdims=True
    )
    part = -jnp.sum(term) / jnp.float32(BATCH)

    @pl.when(pl.program_id(0) == 0)
    def _init():
        out_ref[...] = jnp.zeros((1, 1), jnp.float32)

    out_ref[...] += jnp.reshape(part, (1, 1))


def kernel(yHat, y):
    xT = yHat.T
    y3 = y.reshape(_NBLK, 1, _BB)
    loss = pl.pallas_call(
        _loss_body,
        grid=(_NBLK,),
        in_specs=[
            pl.BlockSpec((CLASSES, _BB), lambda b: (0, b)),
            pl.BlockSpec((1, 1, _BB), lambda b: (b, 0, 0)),
        ],
        out_specs=pl.BlockSpec((1, 1), lambda b: (0, 0)),
        out_shape=jax.ShapeDtypeStruct((1, 1), jnp.float32),
    )(xT, y3)
    return loss[0, 0]
